# baseline (speedup 1.0000x reference)
import functools

import jax
import jax.numpy as jnp
from jax.experimental import pallas as pl
from jax.experimental.pallas import tpu as pltpu

_F32 = jnp.float32
_BF16 = jnp.bfloat16

_P1_KH, _P1_KW, _P1_SW = 4, 3, 3
_P2_KW, _P2_SW = 3, 3


def _body(dims, x_ref, w1_ref, b1_ref, w2_ref, b2_ref,
          wf1_ref, bf1_ref, wf2_ref, bf2_ref, wf3_ref, bf3_ref, o_ref):
    (S, OH1, C1, KW1, HP, OWp1, OW2, KW2, C2, OWp2, P, NU) = dims

    xm = x_ref[...].reshape(S * OWp1, NU * HP)
    acc = jnp.dot(xm, w1_ref[...], preferred_element_type=_F32)

    parts = [acc[:, j * C1:(j + 1) * C1] for j in range(3 * OH1)]
    while len(parts) > 1:
        parts = [jnp.maximum(parts[i], parts[i + 1]) if i + 1 < len(parts)
                 else parts[i] for i in range(0, len(parts), 2)]
    z = parts[0] + b1_ref[...].reshape(1, C1)
    p1 = jnp.maximum(z, 0.0).astype(_BF16).reshape(S, OWp1, C1)

    NV = _P2_SW + KW2 - 1
    p1r = p1.reshape(S, OWp1 // _P2_SW, _P2_SW, C1)
    xc2 = jnp.concatenate(
        [p1r[:, v // _P2_SW:v // _P2_SW + P, v % _P2_SW, :] for v in range(NV)],
        axis=2)
    acc2 = jnp.dot(xc2.reshape(S * P, NV * C1), w2_ref[...],
                   preferred_element_type=_F32)
    q = acc2[:, 0:C2]
    for e in range(1, _P2_SW):
        q = jnp.maximum(q, acc2[:, e * C2:(e + 1) * C2])
    h = jnp.maximum(q + b2_ref[...].reshape(1, C2), 0.0).astype(_BF16)
    h = h.reshape(S, P, C2)

    hf = jnp.concatenate([h[:, p, :] for p in range(P)], axis=1)
    h1 = jnp.dot(hf, wf1_ref[...], preferred_element_type=_F32) + bf1_ref[...]
    h1 = jnp.maximum(h1, 0.0).astype(_BF16)
    h2 = jnp.dot(h1, wf2_ref[...], preferred_element_type=_F32) + bf2_ref[...]
    h2 = jnp.maximum(h2, 0.0).astype(_BF16)
    o_ref[...] = (jnp.dot(h2, wf3_ref[...], preferred_element_type=_F32)
                  + bf3_ref[...])


def _full(shape):
    n = len(shape)
    return pl.BlockSpec(shape, lambda *_: (0,) * n)


@jax.jit
def kernel(x, w1, b1, w2, b2, fc1_w, fc1_b, fc2_w, fc2_b, fc3_w, fc3_b):
    B, Cin, H, W = x.shape
    C1, _, KH1, KW1 = w1.shape
    C2, _, _, KW2 = w2.shape
    FC1, FC2, NC = fc1_w.shape[1], fc2_w.shape[1], fc3_w.shape[1]

    OH1, OW1 = H - KH1 + 1, W - KW1 + 1
    OWp1 = (OW1 - _P1_KW) // _P1_SW + 1
    OW2 = OWp1 - KW2 + 1
    OWp2 = (OW2 - _P2_KW) // _P2_SW + 1
    P = OWp2
    HP = (H + 63) // 64 * 64
    NU = _P1_SW + KW1 - 1
    WT = (W + _P1_SW - 1) // _P1_SW + 1

    S = 256
    while B % S:
        S //= 2

    xpad = jnp.pad(x.reshape(B, Cin * H, W).astype(_BF16),
                   ((0, 0), (0, HP - H), (0, _P1_SW * WT - W)))
    xP = jnp.transpose(xpad.reshape(B, HP, WT, _P1_SW), (0, 3, 2, 1))
    xG = jnp.concatenate(
        [xP[:, u % _P1_SW, u // _P1_SW:u // _P1_SW + OWp1, :]
         for u in range(NU)], axis=2)
    w1r = jnp.transpose(w1[:, 0], (2, 1, 0))
    wpad = jnp.pad(w1r, ((0, NU - KW1), (0, HP - KH1), (0, 0)))
    shifts = [jnp.pad(wpad, ((d, 0), (oh, 0), (0, 0)))[:NU, :HP, :]
              for d in range(_P1_SW) for oh in range(OH1)]
    w1m = jnp.stack(shifts, axis=2).reshape(NU * HP, _P1_SW * OH1 * C1)
    w1m = w1m.astype(_BF16)
    NV = _P2_SW + KW2 - 1
    w2r = jnp.transpose(w2[:, :, 0, :], (2, 1, 0))
    shifts2 = [jnp.pad(w2r, ((e, NV - KW2 - e), (0, 0), (0, 0)))
               for e in range(_P2_SW)]
    w2m = jnp.stack(shifts2, axis=2).reshape(NV * C1, _P2_SW * C2)
    w2m = w2m.astype(_BF16)
    wf1 = jnp.transpose(fc1_w.reshape(C2, P, FC1), (1, 0, 2))
    wf1 = wf1.reshape(P * C2, FC1).astype(_BF16)
    wf2 = fc2_w.astype(_BF16)
    wf3 = fc3_w.astype(_BF16)
    b1r = b1.reshape(1, C1).astype(_F32)
    b2r = b2.reshape(1, C2).astype(_F32)
    bf1 = fc1_b.reshape(1, FC1).astype(_F32)
    bf2 = fc2_b.reshape(1, FC2).astype(_F32)
    bf3 = fc3_b.reshape(1, NC).astype(_F32)

    dims = (S, OH1, C1, KW1, HP, OWp1, OW2, KW2, C2, OWp2, P, NU)
    flops = 2 * B * (OWp1 * NU * HP * _P1_SW * OH1 * C1 + OW2 * KW2 * C1 * C2
                     + P * C2 * FC1 + FC1 * FC2 + FC2 * NC)
    weight_bytes = sum(int(a.size) * a.dtype.itemsize
                       for a in (w1m, w2m, wf1, wf2, wf3, b1r, b2r, bf1, bf2, bf3))
    cost = pl.CostEstimate(
        flops=int(flops), transcendentals=0,
        bytes_accessed=int(xG.size) * 2 + weight_bytes + int(B) * int(NC) * 4)

    out = pl.pallas_call(
        functools.partial(_body, dims),
        out_shape=jax.ShapeDtypeStruct((B, NC), _F32),
        grid=(B // S,),
        in_specs=[
            pl.BlockSpec((S, OWp1, NU * HP), lambda i: (i, 0, 0)),
            _full(w1m.shape), _full(b1r.shape),
            _full(w2m.shape), _full(b2r.shape),
            _full(wf1.shape), _full(bf1.shape),
            _full(wf2.shape), _full(bf2.shape),
            _full(wf3.shape), _full(bf3.shape),
        ],
        out_specs=pl.BlockSpec((S, NC), lambda i: (i, 0)),
        compiler_params=pltpu.CompilerParams(
            dimension_semantics=("arbitrary",),
            vmem_limit_bytes=64 * 1024 * 1024),
        cost_estimate=cost,
    )(xG, w1m, b1r, w2m, b2r, wf1, bf1, wf2, bf2, wf3, bf3)
    return out


# speedup vs baseline: 1.9469x; 1.9469x over previous
import functools

import jax
import jax.numpy as jnp
from jax.experimental import pallas as pl
from jax.experimental.pallas import tpu as pltpu

_F32 = jnp.float32
_BF16 = jnp.bfloat16

_P1_KH, _P1_KW, _P1_SW = 4, 3, 3
_P2_KW, _P2_SW = 3, 3


def _body(dims, x_ref, w1_ref, b1_ref, w2_ref, b2_ref,
          wf1_ref, bf1_ref, wf2_ref, bf2_ref, wf3_ref, bf3_ref, o_ref):
    (S, OH1, C1, KW1, HP, OWp1, OW2, KW2, C2, OWp2, P, NU) = dims

    xv = x_ref[...]
    xc = jnp.concatenate(
        [xv[:, u % 3, u // 3:u // 3 + OWp1, :] for u in range(NU)],
        axis=2)
    xm = xc.reshape(S * OWp1, NU * HP)
    acc = jnp.dot(xm, w1_ref[...], preferred_element_type=_F32)

    parts = [acc[:, j * C1:(j + 1) * C1] for j in range(3 * OH1)]
    while len(parts) > 1:
        parts = [jnp.maximum(parts[i], parts[i + 1]) if i + 1 < len(parts)
                 else parts[i] for i in range(0, len(parts), 2)]
    z = parts[0] + b1_ref[...].reshape(1, C1)
    p1 = jnp.maximum(z, 0.0).astype(_BF16).reshape(S, OWp1, C1)

    NV = _P2_SW + KW2 - 1
    p1r = p1.reshape(S, OWp1 // _P2_SW, _P2_SW, C1)
    xc2 = jnp.concatenate(
        [p1r[:, v // _P2_SW:v // _P2_SW + P, v % _P2_SW, :] for v in range(NV)],
        axis=2)
    acc2 = jnp.dot(xc2.reshape(S * P, NV * C1), w2_ref[...],
                   preferred_element_type=_F32)
    q = acc2[:, 0:C2]
    for e in range(1, _P2_SW):
        q = jnp.maximum(q, acc2[:, e * C2:(e + 1) * C2])
    h = jnp.maximum(q + b2_ref[...].reshape(1, C2), 0.0).astype(_BF16)
    h = h.reshape(S, P, C2)

    hf = jnp.concatenate([h[:, p, :] for p in range(P)], axis=1)
    h1 = jnp.dot(hf, wf1_ref[...], preferred_element_type=_F32) + bf1_ref[...]
    h1 = jnp.maximum(h1, 0.0).astype(_BF16)
    h2 = jnp.dot(h1, wf2_ref[...], preferred_element_type=_F32) + bf2_ref[...]
    h2 = jnp.maximum(h2, 0.0).astype(_BF16)
    o_ref[...] = (jnp.dot(h2, wf3_ref[...], preferred_element_type=_F32)
                  + bf3_ref[...])


def _full(shape):
    n = len(shape)
    return pl.BlockSpec(shape, lambda *_: (0,) * n)


@jax.jit
def kernel(x, w1, b1, w2, b2, fc1_w, fc1_b, fc2_w, fc2_b, fc3_w, fc3_b):
    B, Cin, H, W = x.shape
    C1, _, KH1, KW1 = w1.shape
    C2, _, _, KW2 = w2.shape
    FC1, FC2, NC = fc1_w.shape[1], fc2_w.shape[1], fc3_w.shape[1]

    OH1, OW1 = H - KH1 + 1, W - KW1 + 1
    OWp1 = (OW1 - _P1_KW) // _P1_SW + 1
    OW2 = OWp1 - KW2 + 1
    OWp2 = (OW2 - _P2_KW) // _P2_SW + 1
    P = OWp2
    HP = (H + 63) // 64 * 64
    NU = _P1_SW + KW1 - 1
    WT = (W + _P1_SW - 1) // _P1_SW + 1

    S = 256
    while B % S:
        S //= 2

    xpad = jnp.pad(x.reshape(B, Cin * H, W).astype(_BF16),
                   ((0, 0), (0, HP - H), (0, _P1_SW * WT - W)))
    xP = jnp.transpose(xpad.reshape(B, HP, WT, _P1_SW), (0, 3, 2, 1))
    w1r = jnp.transpose(w1[:, 0], (2, 1, 0))
    wpad = jnp.pad(w1r, ((0, NU - KW1), (0, HP - KH1), (0, 0)))
    shifts = [jnp.pad(wpad, ((d, 0), (oh, 0), (0, 0)))[:NU, :HP, :]
              for d in range(_P1_SW) for oh in range(OH1)]
    w1m = jnp.stack(shifts, axis=2).reshape(NU * HP, _P1_SW * OH1 * C1)
    w1m = w1m.astype(_BF16)
    NV = _P2_SW + KW2 - 1
    w2r = jnp.transpose(w2[:, :, 0, :], (2, 1, 0))
    shifts2 = [jnp.pad(w2r, ((e, NV - KW2 - e), (0, 0), (0, 0)))
               for e in range(_P2_SW)]
    w2m = jnp.stack(shifts2, axis=2).reshape(NV * C1, _P2_SW * C2)
    w2m = w2m.astype(_BF16)
    wf1 = jnp.transpose(fc1_w.reshape(C2, P, FC1), (1, 0, 2))
    wf1 = wf1.reshape(P * C2, FC1).astype(_BF16)
    wf2 = fc2_w.astype(_BF16)
    wf3 = fc3_w.astype(_BF16)
    b1r = b1.reshape(1, C1).astype(_F32)
    b2r = b2.reshape(1, C2).astype(_F32)
    bf1 = fc1_b.reshape(1, FC1).astype(_F32)
    bf2 = fc2_b.reshape(1, FC2).astype(_F32)
    bf3 = fc3_b.reshape(1, NC).astype(_F32)

    dims = (S, OH1, C1, KW1, HP, OWp1, OW2, KW2, C2, OWp2, P, NU)
    flops = 2 * B * (OWp1 * NU * HP * _P1_SW * OH1 * C1 + OW2 * KW2 * C1 * C2
                     + P * C2 * FC1 + FC1 * FC2 + FC2 * NC)
    weight_bytes = sum(int(a.size) * a.dtype.itemsize
                       for a in (w1m, w2m, wf1, wf2, wf3, b1r, b2r, bf1, bf2, bf3))
    cost = pl.CostEstimate(
        flops=int(flops), transcendentals=0,
        bytes_accessed=int(xP.size) * 2 + weight_bytes + int(B) * int(NC) * 4)

    out = pl.pallas_call(
        functools.partial(_body, dims),
        out_shape=jax.ShapeDtypeStruct((B, NC), _F32),
        grid=(B // S,),
        in_specs=[
            pl.BlockSpec((S, _P1_SW, WT, HP), lambda i: (i, 0, 0, 0)),
            _full(w1m.shape), _full(b1r.shape),
            _full(w2m.shape), _full(b2r.shape),
            _full(wf1.shape), _full(bf1.shape),
            _full(wf2.shape), _full(bf2.shape),
            _full(wf3.shape), _full(bf3.shape),
        ],
        out_specs=pl.BlockSpec((S, NC), lambda i: (i, 0)),
        compiler_params=pltpu.CompilerParams(
            dimension_semantics=("arbitrary",),
            vmem_limit_bytes=64 * 1024 * 1024),
        cost_estimate=cost,
    )(xP, w1m, b1r, w2m, b2r, wf1, bf1, wf2, bf2, wf3, bf3)
    return out
